# baseline (speedup 1.0000x reference)
import jax
import jax.numpy as jnp
from jax.experimental import pallas as pl
from jax.experimental.pallas import tpu as pltpu

_EPS = 1e-5
_V = 64
_B = 64
_E = 64
_G = 4
_BF = jnp.bfloat16


def _dt(a, b):
    return jax.lax.dot_general(a.astype(_BF), b.astype(_BF),
                               (((1,), (1,)), ((), ())),
                               preferred_element_type=jnp.float32)


def _mm(a, b):
    return jax.lax.dot_general(a.astype(_BF), b.astype(_BF),
                               (((1,), (0,)), ((), ())),
                               preferred_element_type=jnp.float32)


def _ln(x, w, b):
    m = jnp.mean(x, axis=-1, keepdims=True)
    v = jnp.mean((x - m) ** 2, axis=-1, keepdims=True)
    return (x - m) / jnp.sqrt(v + _EPS) * w + b


def _mlp3(x, W1, b1, W2, b2, W3, b3):
    h = jax.nn.relu(_dt(x, W1) + b1)
    h = jax.nn.relu(_dt(h, W2) + b2)
    return _dt(h, W3) + b3


def _stage_a(w1e_ref, wm_ref, vr_ref, e1_ref, e1n_ref):
    W1e = w1e_ref[...]
    nemb, We2, We3 = wm_ref[0], wm_ref[1], wm_ref[2]
    vr = vr_ref[...]
    edge0, g0 = vr[0:1], vr[1:2]
    b1, b2, b3 = vr[2:3], vr[3:4], vr[4:5]
    lnw, lnb = vr[5:6], vr[6:7]

    VV = _V * _V
    X = jnp.concatenate([
        jnp.broadcast_to(edge0, (VV, _E)),
        jnp.broadcast_to(nemb[:, None, :], (_V, _V, _E)).reshape(VV, _E),
        jnp.broadcast_to(nemb[None, :, :], (_V, _V, _E)).reshape(VV, _E),
        jnp.broadcast_to(g0, (VV, _E)),
    ], axis=1)
    H = jax.nn.relu(_dt(X, W1e) + b1)
    H = jax.nn.relu(_dt(H, We2) + b2)
    E1 = _dt(H, We3) + b3
    e1_ref[...] = E1
    e1n_ref[...] = _ln(edge0 + E1, lnw, lnb)


def _fold_eagg(tm_ref):
    def body(i, acc):
        blk = tm_ref[:, pl.ds(i, 1), :, :]
        for j in range(_V):
            acc = acc + blk[:, 0, j, :]
        return acc
    return jax.lax.fori_loop(0, _V, body, jnp.zeros((_G, _E), jnp.float32))


def _stage_b(adj_ref, et_ref, w1e_ref, w1n_ref, w1g_ref, wm_ref, vr_ref,
             wp_ref, bp_ref, out_ref, stop_ref, tm_ref):
    adj = adj_ref[...].astype(jnp.float32)
    E1 = et_ref[0]
    E1n = et_ref[1]
    W1e = w1e_ref[...]
    W1n = w1n_ref[...]
    W1g = w1g_ref[...]

    nemb = wm_ref[0]
    We2, We3 = wm_ref[1], wm_ref[2]
    Wn2, Wn3 = wm_ref[3], wm_ref[4]
    Wg2, Wg3 = wm_ref[5], wm_ref[6]
    Wq, Wk, Wv, Wo = wm_ref[7], wm_ref[8], wm_ref[9], wm_ref[10]
    S1, S2, S3 = wm_ref[11], wm_ref[12], wm_ref[13]
    R1, R2, R3 = wm_ref[14], wm_ref[15], wm_ref[16]
    T1, T2 = wm_ref[17], wm_ref[18]
    vr = vr_ref[...]
    g0 = vr[0:1]
    be1, be2, be3 = vr[1:2], vr[2:3], vr[3:4]
    bn1, bn2, bn3 = vr[4:5], vr[5:6], vr[6:7]
    bg1, bg2, bg3 = vr[7:8], vr[8:9], vr[9:10]
    lnn_w, lnn_b = vr[10:11], vr[11:12]
    lng_w, lng_b = vr[12:13], vr[13:14]
    lpn_w, lpn_b = vr[14:15], vr[15:16]
    lpg_w, lpg_b = vr[16:17], vr[17:18]
    bs1, bs2, bs3 = vr[18:19], vr[19:20], vr[20:21]
    br1, br2, br3 = vr[21:22], vr[22:23], vr[23:24]
    bt1, bt2 = vr[24:25], vr[25:26]
    T3 = vr[26:27]
    bt3 = vr[27, 0]
    lnorm = vr[27, 1]
    Wp = wp_ref[...]
    bp = bp_ref[...]

    GV = _G * _V
    VV = _V * _V
    E1r = E1.reshape(_V, _V, _E)

    adjT = jnp.swapaxes(adj, 1, 2)
    sagg3 = jnp.zeros((_G, _V, _E), jnp.float32)
    ragg3 = jnp.zeros((_G, _V, _E), jnp.float32)
    for j in range(_V):
        tmp = adj[:, :, j:j + 1] * E1r[:, j, :][None]
        sagg3 = sagg3 + tmp
        tm_ref[:, :, j, :] = tmp
    for i in range(_V):
        ragg3 = ragg3 + adjT[:, :, i:i + 1] * E1r[i][None]
    sagg = sagg3.reshape(GV, _E)
    ragg = ragg3.reshape(GV, _E)
    eagg = _fold_eagg(tm_ref)

    nembG = jnp.broadcast_to(nemb[None], (_G, _V, _E)).reshape(GV, _E)
    Xn = jnp.concatenate([nembG, sagg, ragg,
                          jnp.broadcast_to(g0, (GV, _E))], axis=1)
    h = jax.nn.relu(_dt(Xn, W1n) + bn1)
    h = jax.nn.relu(_dt(h, Wn2) + bn2)
    nn1 = _dt(h, Wn3) + bn3
    nn1r = nn1.reshape(_G, _V, _E)
    nagg = jnp.zeros((_G, _E), jnp.float32)
    for v_ in range(_V):
        nagg = nagg + nn1r[:, v_, :]

    Xg = jnp.concatenate([nagg, eagg, jnp.broadcast_to(g0, (_G, _E))], axis=1)
    gh = jax.nn.relu(_dt(Xg, W1g) + bg1)
    gh = jax.nn.relu(_dt(gh, Wg2) + bg2)
    ng1 = _dt(gh, Wg3) + bg3

    nodes1 = _ln(nembG + nn1, lnn_w, lnn_b)
    g1 = _ln(g0 + ng1, lng_w, lng_b)

    H1s = []
    for g in range(_G):
        n1g = nodes1[g * _V:(g + 1) * _V]
        Xe = jnp.concatenate([
            E1n,
            jnp.broadcast_to(n1g[:, None, :], (_V, _V, _E)).reshape(VV, _E),
            jnp.broadcast_to(n1g[None, :, :], (_V, _V, _E)).reshape(VV, _E),
            jnp.broadcast_to(g1[g:g + 1], (VV, _E)),
        ], axis=1)
        H1s.append(_dt(Xe, W1e))
    H = jax.nn.relu(jnp.concatenate(H1s, axis=0) + be1)
    H = jax.nn.relu(_dt(H, We2) + be2)
    E2 = _dt(H, We3) + be3

    E2v = E2.reshape(_G, _V, _V, _E)
    sagg3 = jnp.zeros((_G, _V, _E), jnp.float32)
    ragg3 = jnp.zeros((_G, _V, _E), jnp.float32)
    for j in range(_V):
        tmp = adj[:, :, j:j + 1] * E2v[:, :, j, :]
        sagg3 = sagg3 + tmp
        tm_ref[:, :, j, :] = tmp
    for i in range(_V):
        ragg3 = ragg3 + adjT[:, :, i:i + 1] * E2v[:, i, :, :]
    sagg2 = sagg3.reshape(GV, _E)
    ragg2 = ragg3.reshape(GV, _E)
    eagg2 = _fold_eagg(tm_ref)

    g1G = jnp.broadcast_to(g1[:, None, :], (_G, _V, _E)).reshape(GV, _E)
    Xn = jnp.concatenate([nodes1, sagg2, ragg2, g1G], axis=1)
    h = jax.nn.relu(_dt(Xn, W1n) + bn1)
    h = jax.nn.relu(_dt(h, Wn2) + bn2)
    nn2 = _dt(h, Wn3) + bn3
    nn2r = nn2.reshape(_G, _V, _E)
    nagg2 = jnp.zeros((_G, _E), jnp.float32)
    for v_ in range(_V):
        nagg2 = nagg2 + nn2r[:, v_, :]

    Xg = jnp.concatenate([nagg2, eagg2, g1], axis=1)
    gh = jax.nn.relu(_dt(Xg, W1g) + bg1)
    gh = jax.nn.relu(_dt(gh, Wg2) + bg2)
    ng2 = _dt(gh, Wg3) + bg3

    nodes2 = _ln(nodes1 + nn2, lnn_w, lnn_b)
    g2 = _ln(g1 + ng2, lng_w, lng_b)

    nf0 = _dt(nodes2, Wp) + bp
    q, k, v = nf0[:, :_E], nf0[:, _E:2 * _E], nf0[:, 2 * _E:]
    qp, kp, vp = _dt(q, Wq), _dt(k, Wk), _dt(v, Wv)
    parts = []
    for hh in range(4):
        sl = slice(16 * hh, 16 * (hh + 1))
        lgs = []
        for g in range(_G):
            gs = slice(g * _V, (g + 1) * _V)
            lgs.append(_dt(qp[gs, sl], kp[gs, sl]) * 0.25)
        lg = jnp.concatenate(lgs, axis=0)
        lg = lg - jnp.max(lg, axis=1, keepdims=True)
        w = jnp.exp(lg)
        w = w / jnp.sum(w, axis=1, keepdims=True)
        parts.append(jnp.concatenate(
            [_mm(w[g * _V:(g + 1) * _V], vp[g * _V:(g + 1) * _V, sl])
             for g in range(_G)], axis=0))
    o = jnp.concatenate(parts, axis=1)
    nf = _ln(_dt(o, Wo), lpn_w, lpn_b)
    gf = _ln(g2, lpg_w, lpg_b)

    s = _mlp3(nf, S1, bs1, S2, bs2, S3, bs3)
    r = _mlp3(nf, R1, br1, R2, br2, R3, br3)
    for g in range(_G):
        gs = slice(g * _V, (g + 1) * _V)
        out_ref[g] = _dt(s[gs], r[gs]) / lnorm

    th = jax.nn.relu(_dt(gf, T1) + bt1)
    th = jax.nn.relu(_dt(th, T2) + bt2)
    stop = (jnp.sum(th.astype(_BF).astype(jnp.float32)
                    * T3.astype(_BF).astype(jnp.float32),
                    axis=1, keepdims=True) + bt3) / lnorm
    stop_ref[...] = jnp.broadcast_to(stop[:, :, None], (_G, 1, 128))


def kernel(adjacency, params):
    p = params
    (W1e, be1), (We2, be2), (We3, be3) = p['edge_mlp']
    (W1n, bn1), (Wn2, bn2), (Wn3, bn3) = p['node_mlp']
    (W1g, bg1), (Wg2, bg2), (Wg3, bg3) = p['global_mlp']
    nemb = p['node_emb']
    edge0 = p['edge_emb'].reshape(_E)
    g0 = p['global_emb'].reshape(_E)
    lne_w, lne_b = p['ln_edges']
    lnn_w, lnn_b = p['ln_nodes']
    lng_w, lng_b = p['ln_globals']
    lpn_w, lpn_b = p['ln_post_nodes']
    lpg_w, lpg_b = p['ln_post_globals']
    Wp, bp = p['attn_proj']
    at = p['attn']
    (S1, bs1), (S2, bs2), (S3, bs3) = p['senders_mlp']
    (R1, br1), (R2, br2), (R3, br3) = p['receivers_mlp']
    (T1, bt1), (T2, bt2), (T3, bt3) = p['stop_mlp']
    lnorm = p['logits_norm']

    wmA = jnp.stack([nemb, We2, We3])
    vrA = jnp.stack([edge0, g0, be1, be2, be3, lne_w, lne_b])

    E1, E1n = pl.pallas_call(
        _stage_a,
        out_shape=(jax.ShapeDtypeStruct((_V * _V, _E), jnp.float32),
                   jax.ShapeDtypeStruct((_V * _V, _E), jnp.float32)),
    )(W1e, wmA, vrA)
    etab = jnp.stack([E1, E1n])

    wmB = jnp.stack([
        nemb, We2, We3, Wn2, Wn3, Wg2, Wg3,
        at['Wq'], at['Wk'], at['Wv'], at['Wo'],
        S1, S2, S3, R1, R2, R3, T1, T2,
    ])
    scal = jnp.zeros((_E,), jnp.float32).at[0].set(bt3[0]).at[1].set(lnorm[0])
    vrB = jnp.stack([
        g0, be1, be2, be3, bn1, bn2, bn3, bg1, bg2, bg3,
        lnn_w, lnn_b, lng_w, lng_b, lpn_w, lpn_b, lpg_w, lpg_b,
        bs1, bs2, bs3, br1, br2, br3, bt1, bt2, T3.reshape(_E), scal,
    ])

    pair_logits, stop_col = pl.pallas_call(
        _stage_b,
        grid=(_B // _G,),
        in_specs=[
            pl.BlockSpec((_G, _V, _V), lambda b: (b, 0, 0)),
            pl.BlockSpec((2, _V * _V, _E), lambda b: (0, 0, 0)),
            pl.BlockSpec((_E, 4 * _E), lambda b: (0, 0)),
            pl.BlockSpec((_E, 4 * _E), lambda b: (0, 0)),
            pl.BlockSpec((_E, 3 * _E), lambda b: (0, 0)),
            pl.BlockSpec((19, _E, _E), lambda b: (0, 0, 0)),
            pl.BlockSpec((28, _E), lambda b: (0, 0)),
            pl.BlockSpec((3 * _E, _E), lambda b: (0, 0)),
            pl.BlockSpec((1, 3 * _E), lambda b: (0, 0)),
        ],
        out_specs=[
            pl.BlockSpec((_G, _V, _V), lambda b: (b, 0, 0)),
            pl.BlockSpec((_G, 1, 128), lambda b: (b, 0, 0)),
        ],
        out_shape=(jax.ShapeDtypeStruct((_B, _V, _V), jnp.float32),
                   jax.ShapeDtypeStruct((_B, 1, 128), jnp.float32)),
        scratch_shapes=[pltpu.VMEM((_G, _V, _V, _E), jnp.float32)],
        compiler_params=pltpu.CompilerParams(
            dimension_semantics=("parallel",)),
    )(adjacency, etab, W1e, W1n, W1g, wmB, vrB, Wp, bp.reshape(1, 3 * _E))

    fwd = jnp.concatenate([pair_logits.reshape(_B, _V * _V),
                           stop_col[:, 0, :1]], axis=1)
    bwd = jnp.zeros((_B, _V * _V + 1), jnp.float32)
    return fwd, bwd
